# Initial kernel scaffold; baseline (speedup 1.0000x reference)
import functools

import jax
import jax.numpy as jnp
from jax import lax
from jax.experimental import pallas as pl
from jax.experimental.pallas import tpu as pltpu
from jax.experimental.pallas import tpu_sc as plsc

NN = 10000
EE = 320000
DD = 128
GG = 256
NPAD = 10240
NSUB = 16
K = 128
CH = 160
EPAD = NSUB * CH * K
RPW = NPAD // NSUB
BLK = 1000
NBLK = NN // BLK


def _sc_fill(ref, nrows, value):
    ncol = ref.shape[1] // 16
    v = jnp.full((16,), value, jnp.float32)

    def row(i, _):
        def col(j, _):
            ref[i, pl.ds(j * 16, 16)] = v
            return 0
        return lax.fori_loop(0, ncol, col, 0)

    lax.fori_loop(0, nrows, row, 0)


def _sc_pass_body(with_deg, h_hbm, src_hbm, dst_hbm, *refs):
    if with_deg:
        agg_hbm, deg_hbm, src_v, dst_v, rows_v, ones_v, agg_s, deg_s, sem = refs
    else:
        agg_hbm, src_v, dst_v, rows_v, ones_v, agg_s, sem = refs
    c = lax.axis_index("c")
    s = lax.axis_index("s")

    pltpu.sync_copy(src_hbm.at[c, s], src_v)
    pltpu.sync_copy(dst_hbm.at[c, s], dst_v)

    _sc_fill(rows_v, K, 0.0)
    for t in range(RPW // K):
        pltpu.sync_copy(rows_v, agg_s.at[pl.ds((s * (RPW // K) + t) * K, K)])
    if with_deg:
        _sc_fill(ones_v, K, 0.0)
        for t in range(RPW // K):
            pltpu.sync_copy(ones_v, deg_s.at[pl.ds((s * (RPW // K) + t) * K, K)])
    _sc_fill(ones_v, K, 1.0)
    plsc.subcore_barrier()

    def chunk(i, _):
        pltpu.async_copy(h_hbm.at[src_v.at[i]], rows_v, sem).wait()
        pltpu.sync_copy(rows_v, agg_s.at[dst_v.at[i]], add=True)
        if with_deg:
            pltpu.sync_copy(ones_v, deg_s.at[dst_v.at[i]], add=True)
        return 0

    lax.fori_loop(0, CH, chunk, 0)
    plsc.subcore_barrier()

    pltpu.sync_copy(agg_s.at[pl.ds(s * RPW, RPW)], agg_hbm.at[c, pl.ds(s * RPW, RPW)])
    if with_deg:
        pltpu.sync_copy(deg_s.at[pl.ds(s * RPW, RPW)], deg_hbm.at[c, pl.ds(s * RPW, RPW)])


def _make_sc_pass(with_deg):
    mesh = plsc.VectorSubcoreMesh(core_axis_name="c", subcore_axis_name="s")
    out_type = [jax.ShapeDtypeStruct((2, NPAD, DD), jnp.float32)]
    scratch = [
        pltpu.VMEM((CH, K), jnp.int32),
        pltpu.VMEM((CH, K), jnp.int32),
        pltpu.VMEM((K, DD), jnp.float32),
        pltpu.VMEM((K, 16), jnp.float32),
        pltpu.VMEM_SHARED((NPAD, DD), jnp.float32),
        pltpu.SemaphoreType.DMA,
    ]
    if with_deg:
        out_type.append(jax.ShapeDtypeStruct((2, NPAD, 16), jnp.float32))
        scratch.insert(5, pltpu.VMEM_SHARED((NPAD, 16), jnp.float32))
    return pl.kernel(
        functools.partial(_sc_pass_body, with_deg),
        out_type=tuple(out_type),
        mesh=mesh,
        scratch_types=scratch,
        name="sc_seg_sum_deg" if with_deg else "sc_seg_sum",
    )


def _tanh_proj_body(x_ref, w_ref, o_ref):
    o_ref[...] = jnp.tanh(
        jnp.dot(x_ref[...], w_ref[...], preferred_element_type=jnp.float32))


def _layer_body(agg_ref, deg_ref, h_ref, w_ref, o_ref):
    a = agg_ref[0]
    d = deg_ref[0]
    invd = 1.0 / jnp.clip(d[:, :1], 1.0)
    out = jnp.dot(a * invd, w_ref[...], preferred_element_type=jnp.float32)
    o_ref[...] = jnp.maximum(out, 0.0) + h_ref[...]


def _layer_readout_body(agg_ref, deg_ref, h_ref, w_ref, b_ref, gsum_ref, cnt_ref):
    c = pl.program_id(0)
    i = pl.program_id(1)
    a = agg_ref[0]
    d = deg_ref[0]
    invd = 1.0 / jnp.clip(d[:, :1], 1.0)
    out = jnp.dot(a * invd, w_ref[...], preferred_element_type=jnp.float32)
    h2 = jnp.maximum(out, 0.0) + h_ref[...]

    b = b_ref[0]
    gid = lax.broadcasted_iota(jnp.int32, (GG, BLK), 0)
    onehot = (gid == b).astype(jnp.float32)
    gblk = jnp.dot(onehot, h2, preferred_element_type=jnp.float32)

    @pl.when(i == 0)
    def _():
        gsum_ref[0] = gblk

    @pl.when(i > 0)
    def _():
        gsum_ref[0] += gblk

    @pl.when(jnp.logical_and(c == 0, i == 0))
    def _():
        cnt_ref[...] = jnp.zeros_like(cnt_ref)

    @pl.when(c == 0)
    def _():
        cnt_ref[0, :] += jnp.sum(onehot, axis=1)


def _loss_body(gsum_ref, cnt_ref, o_ref):
    g = gsum_ref[...]
    cnt = jnp.clip(cnt_ref[0], 1.0)
    g1 = g[0] / cnt[:, None]
    g2 = g[1] / cnt[:, None]
    n1 = jnp.sqrt(jnp.sum(g1 * g1, axis=1, keepdims=True))
    n2 = jnp.sqrt(jnp.sum(g2 * g2, axis=1, keepdims=True))
    z1 = g1 / jnp.clip(n1, 1e-12)
    z2 = g2 / jnp.clip(n2, 1e-12)
    dn = (((1,), (1,)), ((), ()))
    s11 = lax.dot_general(z1, z1, dn, preferred_element_type=jnp.float32)
    s22 = lax.dot_general(z2, z2, dn, preferred_element_type=jnp.float32)
    s12 = lax.dot_general(z1, z2, dn, preferred_element_type=jnp.float32)
    e11 = jnp.exp(s11)
    e22 = jnp.exp(s22)
    e12 = jnp.exp(s12)
    r0 = lax.broadcasted_iota(jnp.int32, (GG, GG), 0)
    r1 = lax.broadcasted_iota(jnp.int32, (GG, GG), 1)
    eye = r0 == r1
    zeros = jnp.zeros((GG, GG), jnp.float32)
    d11 = jnp.sum(jnp.where(eye, e11, zeros), axis=1)
    d22 = jnp.sum(jnp.where(eye, e22, zeros), axis=1)
    d12 = jnp.sum(jnp.where(eye, s12, zeros), axis=1)
    x1 = jnp.sum(e11, axis=1) + jnp.sum(e12, axis=1) - d11
    x2 = jnp.sum(e22, axis=1) + jnp.sum(e12, axis=0) - d22
    loss = (jnp.log(x1) - d12) + (jnp.log(x2) - d12)
    o_ref[0, 0] = jnp.mean(loss)


_sc_pass_a = _make_sc_pass(True)
_sc_pass_b = _make_sc_pass(False)

_tanh_proj = pl.pallas_call(
    _tanh_proj_body,
    grid=(NBLK,),
    in_specs=[
        pl.BlockSpec((BLK, DD), lambda i: (i, 0)),
        pl.BlockSpec((DD, DD), lambda i: (0, 0)),
    ],
    out_specs=pl.BlockSpec((BLK, DD), lambda i: (i, 0)),
    out_shape=jax.ShapeDtypeStruct((NN, DD), jnp.float32),
)

_layer1 = pl.pallas_call(
    _layer_body,
    grid=(2, NBLK),
    in_specs=[
        pl.BlockSpec((1, BLK, DD), lambda c, i: (c, i, 0)),
        pl.BlockSpec((1, BLK, 16), lambda c, i: (c, i, 0)),
        pl.BlockSpec((BLK, DD), lambda c, i: (i, 0)),
        pl.BlockSpec((DD, DD), lambda c, i: (0, 0)),
    ],
    out_specs=pl.BlockSpec((BLK, DD), lambda c, i: (c * NBLK + i, 0)),
    out_shape=jax.ShapeDtypeStruct((2 * NN, DD), jnp.float32),
)

_layer2_readout = pl.pallas_call(
    _layer_readout_body,
    grid=(2, NBLK),
    in_specs=[
        pl.BlockSpec((1, BLK, DD), lambda c, i: (c, i, 0)),
        pl.BlockSpec((1, BLK, 16), lambda c, i: (c, i, 0)),
        pl.BlockSpec((BLK, DD), lambda c, i: (c * NBLK + i, 0)),
        pl.BlockSpec((DD, DD), lambda c, i: (0, 0)),
        pl.BlockSpec((1, 1, BLK), lambda c, i: (i, 0, 0)),
    ],
    out_specs=[
        pl.BlockSpec((1, GG, DD), lambda c, i: (c, 0, 0)),
        pl.BlockSpec((1, GG), lambda c, i: (0, 0)),
    ],
    out_shape=[
        jax.ShapeDtypeStruct((2, GG, DD), jnp.float32),
        jax.ShapeDtypeStruct((1, GG), jnp.float32),
    ],
)

_loss = pl.pallas_call(
    _loss_body,
    in_specs=[
        pl.BlockSpec((2, GG, DD), lambda: (0, 0, 0)),
        pl.BlockSpec((1, GG), lambda: (0, 0)),
    ],
    out_specs=pl.BlockSpec((1, 1), lambda: (0, 0)),
    out_shape=jax.ShapeDtypeStruct((1, 1), jnp.float32),
)


@jax.jit
def kernel(x, batch_vec, edge_index1, edge_index2, W_lm, W1, W2):
    src1 = edge_index1[0].astype(jnp.int32)
    dst1 = edge_index1[1].astype(jnp.int32)
    src2 = edge_index2[0].astype(jnp.int32)
    dst2 = edge_index2[1].astype(jnp.int32)
    npad_e = EPAD - EE
    pad0 = jnp.zeros((npad_e,), jnp.int32)
    padd = jnp.full((npad_e,), NPAD - 1, jnp.int32)
    src_a = jnp.stack([
        jnp.concatenate([src1, pad0]),
        jnp.concatenate([src2, pad0]),
    ]).reshape(2, NSUB, CH, K)
    src_b = jnp.stack([
        jnp.concatenate([src1, pad0]),
        jnp.concatenate([src2 + NN, pad0]),
    ]).reshape(2, NSUB, CH, K)
    dst_p = jnp.stack([
        jnp.concatenate([dst1, padd]),
        jnp.concatenate([dst2, padd]),
    ]).reshape(2, NSUB, CH, K)
    batch3d = batch_vec.astype(jnp.int32).reshape(NBLK, 1, BLK)

    h0 = _tanh_proj(x.astype(jnp.float32), W_lm)
    agg1, deg = _sc_pass_a(h0, src_a, dst_p)
    h1 = _layer1(agg1, deg, h0, W1)
    (agg2,) = _sc_pass_b(h1, src_b, dst_p)
    gsum, cnt = _layer2_readout(agg2, deg, h1, W2, batch3d)
    out = _loss(gsum, cnt)
    return out.reshape(())


# speedup vs baseline: 1.2908x; 1.2908x over previous
import functools

import jax
import jax.numpy as jnp
from jax import lax
from jax.experimental import pallas as pl
from jax.experimental.pallas import tpu as pltpu
from jax.experimental.pallas import tpu_sc as plsc

NN = 10000
EE = 320000
DD = 128
GG = 256
NPAD = 10240
NSUB = 16
K = 128
CH = 160
EPAD = NSUB * CH * K
RPW = NPAD // NSUB
BLK = 1000
NBLK = NN // BLK


def _sc_fill(ref, nrows, value):
    ncol = ref.shape[1] // 16
    v = jnp.full((16,), value, jnp.float32)

    def row(i, _):
        def col(j, _):
            ref[i, pl.ds(j * 16, 16)] = v
            return 0
        return lax.fori_loop(0, ncol, col, 0)

    lax.fori_loop(0, nrows, row, 0)


_SC_MESH = dict(core_axis_name="c", subcore_axis_name="s", num_cores=1,
                num_subcores=NSUB)


def _sc_agg_body(h_hbm, src_hbm, dst_hbm, agg_hbm, src_v, dst_v, rows_v,
                 acc_s, sem):
    s = lax.axis_index("s")

    for set_ in range(2):
        _sc_fill(rows_v, K, 0.0)
        for t in range(RPW // K):
            pltpu.sync_copy(rows_v, acc_s.at[pl.ds(s * RPW + t * K, K)])
        plsc.subcore_barrier()

        def chunk(i, _):
            pltpu.sync_copy(src_hbm.at[set_, s, i], src_v)
            pltpu.sync_copy(dst_hbm.at[set_, s, i], dst_v)
            pltpu.async_copy(h_hbm.at[src_v], rows_v, sem).wait()
            pltpu.sync_copy(rows_v, acc_s.at[dst_v], add=True)
            return 0

        lax.fori_loop(0, CH, chunk, 0)
        plsc.subcore_barrier()

        pltpu.sync_copy(acc_s.at[pl.ds(s * RPW, RPW)],
                        agg_hbm.at[set_, pl.ds(s * RPW, RPW)])
        plsc.subcore_barrier()


def _sc_deg_body(dst_hbm, deg_hbm, dst_v, ones_v, deg_s):
    s = lax.axis_index("s")
    for set_ in range(2):
        _sc_fill(ones_v, K, 0.0)
        for t in range(RPW // K):
            pltpu.sync_copy(ones_v, deg_s.at[pl.ds(s * RPW + t * K, K)])
        _sc_fill(ones_v, K, 1.0)
        plsc.subcore_barrier()

        def chunk(i, _):
            pltpu.sync_copy(dst_hbm.at[set_, s, i], dst_v)
            pltpu.sync_copy(ones_v, deg_s.at[dst_v], add=True)
            return 0

        lax.fori_loop(0, CH, chunk, 0)
        plsc.subcore_barrier()
        pltpu.sync_copy(deg_s.at[pl.ds(s * RPW, RPW)],
                        deg_hbm.at[set_, pl.ds(s * RPW, RPW)])
        plsc.subcore_barrier()


@functools.lru_cache(maxsize=None)
def _make_sc_agg(name):
    return pl.kernel(
        _sc_agg_body,
        out_type=jax.ShapeDtypeStruct((2, NPAD, DD), jnp.float32),
        mesh=plsc.VectorSubcoreMesh(**_SC_MESH),
        scratch_types=[
            pltpu.VMEM((K,), jnp.int32),
            pltpu.VMEM((K,), jnp.int32),
            pltpu.VMEM((K, DD), jnp.float32),
            pltpu.VMEM_SHARED((NPAD, DD), jnp.float32),
            pltpu.SemaphoreType.DMA,
        ],
        name=name,
    )


@functools.lru_cache(maxsize=None)
def _make_sc_deg():
    return pl.kernel(
        _sc_deg_body,
        out_type=jax.ShapeDtypeStruct((2, NPAD, DD), jnp.float32),
        mesh=plsc.VectorSubcoreMesh(**_SC_MESH),
        scratch_types=[
            pltpu.VMEM((K,), jnp.int32),
            pltpu.VMEM((K, DD), jnp.float32),
            pltpu.VMEM_SHARED((NPAD, DD), jnp.float32),
        ],
        name="sc_deg",
    )


def _tanh_proj_body(x_ref, w_ref, o_ref):
    o_ref[...] = jnp.tanh(
        jnp.dot(x_ref[...], w_ref[...], preferred_element_type=jnp.float32))


def _layer_body(agg_ref, deg_ref, h_ref, w_ref, o_ref):
    a = agg_ref[0]
    d = deg_ref[0]
    invd = 1.0 / jnp.clip(d[:, :1], 1.0)
    out = jnp.dot(a * invd, w_ref[...], preferred_element_type=jnp.float32)
    o_ref[...] = jnp.maximum(out, 0.0) + h_ref[...]


def _layer_readout_body(agg_ref, deg_ref, h_ref, w_ref, b_ref, gsum_ref, cnt_ref):
    c = pl.program_id(0)
    i = pl.program_id(1)
    a = agg_ref[0]
    d = deg_ref[0]
    invd = 1.0 / jnp.clip(d[:, :1], 1.0)
    out = jnp.dot(a * invd, w_ref[...], preferred_element_type=jnp.float32)
    h2 = jnp.maximum(out, 0.0) + h_ref[...]

    b = b_ref[0]
    gid = lax.broadcasted_iota(jnp.int32, (GG, BLK), 0)
    onehot = (gid == b).astype(jnp.float32)
    gblk = jnp.dot(onehot, h2, preferred_element_type=jnp.float32)

    @pl.when(i == 0)
    def _():
        gsum_ref[0] = gblk

    @pl.when(i > 0)
    def _():
        gsum_ref[0] += gblk

    @pl.when(jnp.logical_and(c == 0, i == 0))
    def _():
        cnt_ref[...] = jnp.zeros_like(cnt_ref)

    @pl.when(c == 0)
    def _():
        cnt_ref[0, :] += jnp.sum(onehot, axis=1)


def _loss_body(gsum_ref, cnt_ref, o_ref):
    g = gsum_ref[...]
    cnt = jnp.clip(cnt_ref[0], 1.0)
    g1 = g[0] / cnt[:, None]
    g2 = g[1] / cnt[:, None]
    n1 = jnp.sqrt(jnp.sum(g1 * g1, axis=1, keepdims=True))
    n2 = jnp.sqrt(jnp.sum(g2 * g2, axis=1, keepdims=True))
    z1 = g1 / jnp.clip(n1, 1e-12)
    z2 = g2 / jnp.clip(n2, 1e-12)
    dn = (((1,), (1,)), ((), ()))
    s11 = lax.dot_general(z1, z1, dn, preferred_element_type=jnp.float32)
    s22 = lax.dot_general(z2, z2, dn, preferred_element_type=jnp.float32)
    s12 = lax.dot_general(z1, z2, dn, preferred_element_type=jnp.float32)
    e11 = jnp.exp(s11)
    e22 = jnp.exp(s22)
    e12 = jnp.exp(s12)
    r0 = lax.broadcasted_iota(jnp.int32, (GG, GG), 0)
    r1 = lax.broadcasted_iota(jnp.int32, (GG, GG), 1)
    eye = r0 == r1
    zeros = jnp.zeros((GG, GG), jnp.float32)
    d11 = jnp.sum(jnp.where(eye, e11, zeros), axis=1)
    d22 = jnp.sum(jnp.where(eye, e22, zeros), axis=1)
    d12 = jnp.sum(jnp.where(eye, s12, zeros), axis=1)
    x1 = jnp.sum(e11, axis=1) + jnp.sum(e12, axis=1) - d11
    x2 = jnp.sum(e22, axis=1) + jnp.sum(e12, axis=0) - d22
    loss = (jnp.log(x1) - d12) + (jnp.log(x2) - d12)
    o_ref[...] = jnp.mean(loss)[None, None]


_tanh_proj = pl.pallas_call(
    _tanh_proj_body,
    grid=(NBLK,),
    in_specs=[
        pl.BlockSpec((BLK, DD), lambda i: (i, 0)),
        pl.BlockSpec((DD, DD), lambda i: (0, 0)),
    ],
    out_specs=pl.BlockSpec((BLK, DD), lambda i: (i, 0)),
    out_shape=jax.ShapeDtypeStruct((NN, DD), jnp.float32),
)

_layer1 = pl.pallas_call(
    _layer_body,
    grid=(2, NBLK),
    in_specs=[
        pl.BlockSpec((1, BLK, DD), lambda c, i: (c, i, 0)),
        pl.BlockSpec((1, BLK, DD), lambda c, i: (c, i, 0)),
        pl.BlockSpec((BLK, DD), lambda c, i: (i, 0)),
        pl.BlockSpec((DD, DD), lambda c, i: (0, 0)),
    ],
    out_specs=pl.BlockSpec((BLK, DD), lambda c, i: (c * NBLK + i, 0)),
    out_shape=jax.ShapeDtypeStruct((2 * NN, DD), jnp.float32),
)

_layer2_readout = pl.pallas_call(
    _layer_readout_body,
    grid=(2, NBLK),
    in_specs=[
        pl.BlockSpec((1, BLK, DD), lambda c, i: (c, i, 0)),
        pl.BlockSpec((1, BLK, DD), lambda c, i: (c, i, 0)),
        pl.BlockSpec((BLK, DD), lambda c, i: (c * NBLK + i, 0)),
        pl.BlockSpec((DD, DD), lambda c, i: (0, 0)),
        pl.BlockSpec((1, 1, BLK), lambda c, i: (i, 0, 0)),
    ],
    out_specs=[
        pl.BlockSpec((1, GG, DD), lambda c, i: (c, 0, 0)),
        pl.BlockSpec((1, GG), lambda c, i: (0, 0)),
    ],
    out_shape=[
        jax.ShapeDtypeStruct((2, GG, DD), jnp.float32),
        jax.ShapeDtypeStruct((1, GG), jnp.float32),
    ],
)

_loss = pl.pallas_call(
    _loss_body,
    in_specs=[
        pl.BlockSpec((2, GG, DD), lambda: (0, 0, 0)),
        pl.BlockSpec((1, GG), lambda: (0, 0)),
    ],
    out_specs=pl.BlockSpec((1, 1), lambda: (0, 0)),
    out_shape=jax.ShapeDtypeStruct((1, 1), jnp.float32),
)


@jax.jit
def kernel(x, batch_vec, edge_index1, edge_index2, W_lm, W1, W2):
    src1 = edge_index1[0].astype(jnp.int32)
    dst1 = edge_index1[1].astype(jnp.int32)
    src2 = edge_index2[0].astype(jnp.int32)
    dst2 = edge_index2[1].astype(jnp.int32)
    npad_e = EPAD - EE
    pad0 = jnp.zeros((npad_e,), jnp.int32)
    padd = jnp.full((npad_e,), NPAD - 1, jnp.int32)
    srcs = jnp.stack([
        jnp.concatenate([src1, pad0]),
        jnp.concatenate([src2, pad0]),
    ])
    src_a = srcs.reshape(2, NSUB, CH, K)
    set_off = jnp.array([0, NN], jnp.int32)[:, None]
    src_b = (srcs + set_off).reshape(2, NSUB, CH, K)
    dst_p = jnp.stack([
        jnp.concatenate([dst1, padd]),
        jnp.concatenate([dst2, padd]),
    ]).reshape(2, NSUB, CH, K)
    batch3d = batch_vec.astype(jnp.int32).reshape(NBLK, 1, BLK)

    deg = _make_sc_deg()(dst_p)
    h0 = _tanh_proj(x.astype(jnp.float32), W_lm)
    agg1 = _make_sc_agg("sc_agg_a")(h0, src_a, dst_p)
    h1 = _layer1(agg1, deg, h0, W1)
    agg2 = _make_sc_agg("sc_agg_b")(h1, src_b, dst_p)
    gsum, cnt = _layer2_readout(agg2, deg, h1, W2, batch3d)
    out = _loss(gsum, cnt)
    return out.reshape(())


# speedup vs baseline: 1.8598x; 1.4408x over previous
import functools

import jax
import jax.numpy as jnp
from jax import lax
from jax.experimental import pallas as pl
from jax.experimental.pallas import tpu as pltpu
from jax.experimental.pallas import tpu_sc as plsc

NN = 10000
EE = 320000
DD = 128
GG = 256
NPAD = 10240
NSUB = 16
K = 128
CH = 160
KD = 128
CHD = 160
EPAD = NSUB * CH * K
RPW = NPAD // NSUB
BLK = 1000
NBLK = NN // BLK


def _sc_fill(ref, nrows, value):
    ncol = ref.shape[1] // 16
    v = jnp.full((16,), value, jnp.float32)

    def row(i, _):
        def col(j, _):
            ref[i, pl.ds(j * 16, 16)] = v
            return 0
        return lax.fori_loop(0, ncol, col, 0)

    lax.fori_loop(0, nrows, row, 0)


_SC_MESH = dict(core_axis_name="c", subcore_axis_name="s", num_cores=1,
                num_subcores=NSUB)


ZCH = 128


def _sc_agg_body(h_hbm, sd_hbm, agg_hbm, sd0, sd1, sd2, sd3, rows0, rows1,
                 acc_s, gsem, ssem, isem):
    s = lax.axis_index("s")
    sdb = (sd0, sd1, sd2, sd3)
    rows = (rows0, rows1)

    for set_ in range(2):
        _sc_fill(rows[0], K, 0.0)
        for t in range(RPW // K):
            pltpu.sync_copy(rows[0], acc_s.at[pl.ds(s * RPW + t * K, K)])
        plsc.subcore_barrier()

        pltpu.sync_copy(sd_hbm.at[set_, s, 0], sdb[0])
        pltpu.sync_copy(sd_hbm.at[set_, s, 1], sdb[1])
        pltpu.async_copy(sd_hbm.at[set_, s, 2], sdb[2], isem)
        pltpu.async_copy(h_hbm.at[sdb[0].at[0]], rows[0], gsem)
        pltpu.async_copy(h_hbm.at[sdb[1].at[0]], rows[1], gsem)

        def outer(t, _):
            for b in range(4):
                i = 4 * t + b
                rb = rows[b % 2]
                sb = sdb[b]
                pltpu.make_async_copy(h_hbm.at[sb.at[0]], rb, gsem).wait()
                pltpu.sync_copy(rb, acc_s.at[sb.at[1]], add=True)

                @pl.when(i + 3 < CH)
                def _():
                    pltpu.async_copy(sd_hbm.at[set_, s, i + 3],
                                     sdb[(b + 3) % 4], isem)

                @pl.when(i + 2 < CH)
                def _():
                    pltpu.make_async_copy(sd_hbm.at[set_, s, 0],
                                          sdb[(b + 2) % 4], isem).wait()
                    pltpu.async_copy(h_hbm.at[sdb[(b + 2) % 4].at[0]],
                                     rb, gsem)
            return 0

        lax.fori_loop(0, CH // 4, outer, 0)
        plsc.subcore_barrier()

        pltpu.sync_copy(acc_s.at[pl.ds(s * RPW, RPW)],
                        agg_hbm.at[set_, pl.ds(s * RPW, RPW)])
        plsc.subcore_barrier()


def _sc_deg_body(dd_hbm, deg_hbm, dst0, dst1, ones_v, zrow_v, deg_s, isem):
    s = lax.axis_index("s")
    dstb = (dst0, dst1)
    _sc_fill(ones_v, KD, 1.0)
    _sc_fill(zrow_v, ZCH, 0.0)
    for set_ in range(2):
        for t in range(RPW // ZCH):
            pltpu.sync_copy(zrow_v, deg_s.at[pl.ds(s * RPW + t * ZCH, ZCH)])
        plsc.subcore_barrier()

        pltpu.sync_copy(dd_hbm.at[set_, s, 0], dstb[0])
        pltpu.async_copy(dd_hbm.at[set_, s, 1], dstb[1], isem)

        def outer(t, _):
            for b in range(2):
                i = 2 * t + b
                db = dstb[b]

                @pl.when(i > 0)
                def _():
                    pltpu.make_async_copy(dd_hbm.at[set_, s, 0], db,
                                          isem).wait()
                pltpu.sync_copy(ones_v, deg_s.at[db], add=True)

                @pl.when(i + 2 < CHD)
                def _():
                    pltpu.async_copy(dd_hbm.at[set_, s, i + 2], db, isem)
            return 0

        lax.fori_loop(0, CHD // 2, outer, 0)
        plsc.subcore_barrier()
        pltpu.sync_copy(deg_s.at[pl.ds(s * RPW, RPW)],
                        deg_hbm.at[set_, pl.ds(s * RPW, RPW)])
        plsc.subcore_barrier()


@functools.lru_cache(maxsize=None)
def _make_sc_agg(name):
    return pl.kernel(
        _sc_agg_body,
        out_type=jax.ShapeDtypeStruct((2, NPAD, DD), jnp.float32),
        mesh=plsc.VectorSubcoreMesh(**_SC_MESH),
        scratch_types=[
            pltpu.VMEM((2, K), jnp.int32),
            pltpu.VMEM((2, K), jnp.int32),
            pltpu.VMEM((2, K), jnp.int32),
            pltpu.VMEM((2, K), jnp.int32),
            pltpu.VMEM((K, DD), jnp.float32),
            pltpu.VMEM((K, DD), jnp.float32),
            pltpu.VMEM_SHARED((NPAD, DD), jnp.float32),
            pltpu.SemaphoreType.DMA,
            pltpu.SemaphoreType.DMA,
            pltpu.SemaphoreType.DMA,
        ],
        name=name,
    )


@functools.lru_cache(maxsize=None)
def _make_sc_deg():
    return pl.kernel(
        _sc_deg_body,
        out_type=jax.ShapeDtypeStruct((2, NPAD, DD), jnp.float32),
        mesh=plsc.VectorSubcoreMesh(**_SC_MESH),
        scratch_types=[
            pltpu.VMEM((KD,), jnp.int32),
            pltpu.VMEM((KD,), jnp.int32),
            pltpu.VMEM((KD, DD), jnp.float32),
            pltpu.VMEM((ZCH, DD), jnp.float32),
            pltpu.VMEM_SHARED((NPAD, DD), jnp.float32),
            pltpu.SemaphoreType.DMA,
        ],
        name="sc_deg",
    )


def _tanh_proj_body(x_ref, w_ref, o_ref):
    o_ref[...] = jnp.tanh(
        jnp.dot(x_ref[...], w_ref[...], preferred_element_type=jnp.float32))


def _layer_body(agg_ref, deg_ref, h_ref, w_ref, o_ref):
    a = agg_ref[0]
    d = deg_ref[0]
    invd = 1.0 / jnp.clip(d[:, :1], 1.0)
    out = jnp.dot(a * invd, w_ref[...], preferred_element_type=jnp.float32)
    o_ref[...] = jnp.maximum(out, 0.0) + h_ref[...]


def _layer_readout_body(agg_ref, deg_ref, h_ref, w_ref, b_ref, gsum_ref, cnt_ref):
    c = pl.program_id(0)
    i = pl.program_id(1)
    a = agg_ref[0]
    d = deg_ref[0]
    invd = 1.0 / jnp.clip(d[:, :1], 1.0)
    out = jnp.dot(a * invd, w_ref[...], preferred_element_type=jnp.float32)
    h2 = jnp.maximum(out, 0.0) + h_ref[...]

    b = b_ref[0]
    gid = lax.broadcasted_iota(jnp.int32, (GG, BLK), 0)
    onehot = (gid == b).astype(jnp.float32)
    gblk = jnp.dot(onehot, h2, preferred_element_type=jnp.float32)

    @pl.when(i == 0)
    def _():
        gsum_ref[0] = gblk

    @pl.when(i > 0)
    def _():
        gsum_ref[0] += gblk

    @pl.when(jnp.logical_and(c == 0, i == 0))
    def _():
        cnt_ref[...] = jnp.zeros_like(cnt_ref)

    @pl.when(c == 0)
    def _():
        cnt_ref[0, :] += jnp.sum(onehot, axis=1)


def _loss_body(gsum_ref, cnt_ref, o_ref):
    g = gsum_ref[...]
    cnt = jnp.clip(cnt_ref[0], 1.0)
    g1 = g[0] / cnt[:, None]
    g2 = g[1] / cnt[:, None]
    n1 = jnp.sqrt(jnp.sum(g1 * g1, axis=1, keepdims=True))
    n2 = jnp.sqrt(jnp.sum(g2 * g2, axis=1, keepdims=True))
    z1 = g1 / jnp.clip(n1, 1e-12)
    z2 = g2 / jnp.clip(n2, 1e-12)
    dn = (((1,), (1,)), ((), ()))
    s11 = lax.dot_general(z1, z1, dn, preferred_element_type=jnp.float32)
    s22 = lax.dot_general(z2, z2, dn, preferred_element_type=jnp.float32)
    s12 = lax.dot_general(z1, z2, dn, preferred_element_type=jnp.float32)
    e11 = jnp.exp(s11)
    e22 = jnp.exp(s22)
    e12 = jnp.exp(s12)
    r0 = lax.broadcasted_iota(jnp.int32, (GG, GG), 0)
    r1 = lax.broadcasted_iota(jnp.int32, (GG, GG), 1)
    eye = r0 == r1
    zeros = jnp.zeros((GG, GG), jnp.float32)
    d11 = jnp.sum(jnp.where(eye, e11, zeros), axis=1)
    d22 = jnp.sum(jnp.where(eye, e22, zeros), axis=1)
    d12 = jnp.sum(jnp.where(eye, s12, zeros), axis=1)
    x1 = jnp.sum(e11, axis=1) + jnp.sum(e12, axis=1) - d11
    x2 = jnp.sum(e22, axis=1) + jnp.sum(e12, axis=0) - d22
    loss = (jnp.log(x1) - d12) + (jnp.log(x2) - d12)
    o_ref[...] = jnp.mean(loss)[None, None]


_tanh_proj = pl.pallas_call(
    _tanh_proj_body,
    grid=(NBLK,),
    in_specs=[
        pl.BlockSpec((BLK, DD), lambda i: (i, 0)),
        pl.BlockSpec((DD, DD), lambda i: (0, 0)),
    ],
    out_specs=pl.BlockSpec((BLK, DD), lambda i: (i, 0)),
    out_shape=jax.ShapeDtypeStruct((NN, DD), jnp.float32),
)

_layer1 = pl.pallas_call(
    _layer_body,
    grid=(2, NBLK),
    in_specs=[
        pl.BlockSpec((1, BLK, DD), lambda c, i: (c, i, 0)),
        pl.BlockSpec((1, BLK, DD), lambda c, i: (c, i, 0)),
        pl.BlockSpec((BLK, DD), lambda c, i: (i, 0)),
        pl.BlockSpec((DD, DD), lambda c, i: (0, 0)),
    ],
    out_specs=pl.BlockSpec((BLK, DD), lambda c, i: (c * NBLK + i, 0)),
    out_shape=jax.ShapeDtypeStruct((2 * NN, DD), jnp.float32),
)

_layer2_readout = pl.pallas_call(
    _layer_readout_body,
    grid=(2, NBLK),
    in_specs=[
        pl.BlockSpec((1, BLK, DD), lambda c, i: (c, i, 0)),
        pl.BlockSpec((1, BLK, DD), lambda c, i: (c, i, 0)),
        pl.BlockSpec((BLK, DD), lambda c, i: (c * NBLK + i, 0)),
        pl.BlockSpec((DD, DD), lambda c, i: (0, 0)),
        pl.BlockSpec((1, 1, BLK), lambda c, i: (i, 0, 0)),
    ],
    out_specs=[
        pl.BlockSpec((1, GG, DD), lambda c, i: (c, 0, 0)),
        pl.BlockSpec((1, GG), lambda c, i: (0, 0)),
    ],
    out_shape=[
        jax.ShapeDtypeStruct((2, GG, DD), jnp.float32),
        jax.ShapeDtypeStruct((1, GG), jnp.float32),
    ],
)

_loss = pl.pallas_call(
    _loss_body,
    in_specs=[
        pl.BlockSpec((2, GG, DD), lambda: (0, 0, 0)),
        pl.BlockSpec((1, GG), lambda: (0, 0)),
    ],
    out_specs=pl.BlockSpec((1, 1), lambda: (0, 0)),
    out_shape=jax.ShapeDtypeStruct((1, 1), jnp.float32),
)


@jax.jit
def kernel(x, batch_vec, edge_index1, edge_index2, W_lm, W1, W2):
    src1 = edge_index1[0].astype(jnp.int32)
    dst1 = edge_index1[1].astype(jnp.int32)
    src2 = edge_index2[0].astype(jnp.int32)
    dst2 = edge_index2[1].astype(jnp.int32)
    npad_e = EPAD - EE
    pad0 = jnp.zeros((npad_e,), jnp.int32)
    padd = jnp.full((npad_e,), NPAD - 1, jnp.int32)
    srcs = jnp.stack([
        jnp.concatenate([src1, pad0]),
        jnp.concatenate([src2, pad0]),
    ])
    set_off = jnp.array([0, NN], jnp.int32)[:, None]
    dsts = jnp.stack([
        jnp.concatenate([dst1, padd]),
        jnp.concatenate([dst2, padd]),
    ])
    sd_a = jnp.stack([srcs.reshape(2, NSUB, CH, K),
                      dsts.reshape(2, NSUB, CH, K)], axis=3)
    sd_b = jnp.stack([(srcs + set_off).reshape(2, NSUB, CH, K),
                      dsts.reshape(2, NSUB, CH, K)], axis=3)
    dd = dsts.reshape(2, NSUB, CHD, KD)
    batch3d = batch_vec.astype(jnp.int32).reshape(NBLK, 1, BLK)

    deg = _make_sc_deg()(dd)
    h0 = _tanh_proj(x.astype(jnp.float32), W_lm)
    agg1 = _make_sc_agg("sc_agg_a")(h0, sd_a)
    h1 = _layer1(agg1, deg, h0, W1)
    agg2 = _make_sc_agg("sc_agg_b")(h1, sd_b)
    gsum, cnt = _layer2_readout(agg2, deg, h1, W2, batch3d)
    out = _loss(gsum, cnt)
    return out.reshape(())


# speedup vs baseline: 2.0731x; 1.1147x over previous
import functools

import jax
import jax.numpy as jnp
from jax import lax
from jax.experimental import pallas as pl
from jax.experimental.pallas import tpu as pltpu
from jax.experimental.pallas import tpu_sc as plsc

NN = 10000
EE = 320000
DD = 128
GG = 256
NPAD = 10240
NSUB = 16
K = 128
CH = 160
KD = 80
CHD = 256
EPAD = NSUB * CH * K
RPW = NPAD // NSUB
BLK = 1000
NBLK = NN // BLK


def _sc_fill(ref, nrows, value):
    ncol = ref.shape[1] // 16
    v = jnp.full((16,), value, jnp.float32)

    def row(i, _):
        def col(j, _):
            ref[i, pl.ds(j * 16, 16)] = v
            return 0
        return lax.fori_loop(0, ncol, col, 0)

    lax.fori_loop(0, nrows, row, 0)


_SC_MESH = dict(core_axis_name="c", subcore_axis_name="s", num_cores=1,
                num_subcores=NSUB)


ZCH = 128


def _sc_agg_body(h_hbm, sd_hbm, agg_hbm, sd0, sd1, sd2, sd3, rows0, rows1,
                 acc_s, gsem, ssem, isem):
    s = lax.axis_index("s")
    sdb = (sd0, sd1, sd2, sd3)
    rows = (rows0, rows1)

    for set_ in range(2):
        _sc_fill(rows[0], K, 0.0)
        for t in range(RPW // K):
            pltpu.sync_copy(rows[0], acc_s.at[pl.ds(s * RPW + t * K, K)])
        plsc.subcore_barrier()

        pltpu.sync_copy(sd_hbm.at[set_, s, 0], sdb[0])
        pltpu.sync_copy(sd_hbm.at[set_, s, 1], sdb[1])
        pltpu.async_copy(sd_hbm.at[set_, s, 2], sdb[2], isem)
        pltpu.async_copy(h_hbm.at[sdb[0].at[0]], rows[0], gsem)
        pltpu.async_copy(h_hbm.at[sdb[1].at[0]], rows[1], gsem)

        def outer(t, _):
            for b in range(4):
                i = 4 * t + b
                rb = rows[b % 2]
                sb = sdb[b]
                pltpu.make_async_copy(h_hbm.at[sb.at[0]], rb, gsem).wait()
                pltpu.sync_copy(rb, acc_s.at[sb.at[1]], add=True)

                @pl.when(i + 3 < CH)
                def _():
                    pltpu.async_copy(sd_hbm.at[set_, s, i + 3],
                                     sdb[(b + 3) % 4], isem)

                @pl.when(i + 2 < CH)
                def _():
                    pltpu.make_async_copy(sd_hbm.at[set_, s, 0],
                                          sdb[(b + 2) % 4], isem).wait()
                    pltpu.async_copy(h_hbm.at[sdb[(b + 2) % 4].at[0]],
                                     rb, gsem)
            return 0

        lax.fori_loop(0, CH // 4, outer, 0)
        plsc.subcore_barrier()

        pltpu.sync_copy(acc_s.at[pl.ds(s * RPW, RPW)],
                        agg_hbm.at[set_, pl.ds(s * RPW, RPW)])
        plsc.subcore_barrier()


def _sc_deg_body(dd_hbm, deg_hbm, dst0, dst1, ones_v, zrow_v, deg_s, isem):
    s = lax.axis_index("s")
    dstb = (dst0, dst1)
    _sc_fill(ones_v, KD, 1.0)
    _sc_fill(zrow_v, ZCH, 0.0)
    for set_ in range(2):
        for t in range(RPW // ZCH):
            pltpu.sync_copy(zrow_v, deg_s.at[pl.ds(s * RPW + t * ZCH, ZCH)])
        plsc.subcore_barrier()

        pltpu.sync_copy(dd_hbm.at[set_, s, 0], dstb[0])
        pltpu.async_copy(dd_hbm.at[set_, s, 1], dstb[1], isem)

        def outer(t, _):
            for b in range(2):
                i = 2 * t + b
                db = dstb[b]

                @pl.when(i > 0)
                def _():
                    pltpu.make_async_copy(dd_hbm.at[set_, s, 0], db,
                                          isem).wait()
                pltpu.sync_copy(ones_v, deg_s.at[db], add=True)

                @pl.when(i + 2 < CHD)
                def _():
                    pltpu.async_copy(dd_hbm.at[set_, s, i + 2], db, isem)
            return 0

        lax.fori_loop(0, CHD // 2, outer, 0)
        plsc.subcore_barrier()
        pltpu.sync_copy(deg_s.at[pl.ds(s * RPW, RPW)],
                        deg_hbm.at[set_, pl.ds(s * RPW, RPW)])
        plsc.subcore_barrier()


@functools.lru_cache(maxsize=None)
def _make_sc_agg(name):
    return pl.kernel(
        _sc_agg_body,
        out_type=jax.ShapeDtypeStruct((2, NPAD, DD), jnp.float32),
        mesh=plsc.VectorSubcoreMesh(**_SC_MESH),
        scratch_types=[
            pltpu.VMEM((2, K), jnp.int32),
            pltpu.VMEM((2, K), jnp.int32),
            pltpu.VMEM((2, K), jnp.int32),
            pltpu.VMEM((2, K), jnp.int32),
            pltpu.VMEM((K, DD), jnp.float32),
            pltpu.VMEM((K, DD), jnp.float32),
            pltpu.VMEM_SHARED((NPAD, DD), jnp.float32),
            pltpu.SemaphoreType.DMA,
            pltpu.SemaphoreType.DMA,
            pltpu.SemaphoreType.DMA,
        ],
        name=name,
    )


@functools.lru_cache(maxsize=None)
def _make_sc_deg():
    return pl.kernel(
        _sc_deg_body,
        out_type=jax.ShapeDtypeStruct((2, NPAD, DD), jnp.float32),
        mesh=plsc.VectorSubcoreMesh(**_SC_MESH),
        scratch_types=[
            pltpu.VMEM((KD,), jnp.int32),
            pltpu.VMEM((KD,), jnp.int32),
            pltpu.VMEM((KD, DD), jnp.float32),
            pltpu.VMEM((ZCH, DD), jnp.float32),
            pltpu.VMEM_SHARED((NPAD, DD), jnp.float32),
            pltpu.SemaphoreType.DMA,
        ],
        name="sc_deg",
    )


def _tanh_proj_body(x_ref, w_ref, o_ref):
    o_ref[...] = jnp.tanh(
        jnp.dot(x_ref[...], w_ref[...], preferred_element_type=jnp.float32))


def _layer_body(agg_ref, deg_ref, h_ref, w_ref, o_ref):
    a = agg_ref[0]
    d = deg_ref[0]
    invd = 1.0 / jnp.clip(d[:, :1], 1.0)
    out = jnp.dot(a * invd, w_ref[...], preferred_element_type=jnp.float32)
    o_ref[...] = jnp.maximum(out, 0.0) + h_ref[...]


def _layer_readout_body(agg_ref, deg_ref, h_ref, w_ref, b_ref, gsum_ref, cnt_ref):
    c = pl.program_id(0)
    i = pl.program_id(1)
    a = agg_ref[0]
    d = deg_ref[0]
    invd = 1.0 / jnp.clip(d[:, :1], 1.0)
    out = jnp.dot(a * invd, w_ref[...], preferred_element_type=jnp.float32)
    h2 = jnp.maximum(out, 0.0) + h_ref[...]

    b = b_ref[0]
    gid = lax.broadcasted_iota(jnp.int32, (GG, BLK), 0)
    onehot = (gid == b).astype(jnp.float32)
    gblk = jnp.dot(onehot, h2, preferred_element_type=jnp.float32)

    @pl.when(i == 0)
    def _():
        gsum_ref[0] = gblk

    @pl.when(i > 0)
    def _():
        gsum_ref[0] += gblk

    @pl.when(jnp.logical_and(c == 0, i == 0))
    def _():
        cnt_ref[...] = jnp.zeros_like(cnt_ref)

    @pl.when(c == 0)
    def _():
        cnt_ref[0, :] += jnp.sum(onehot, axis=1)


def _loss_body(gsum_ref, cnt_ref, o_ref):
    g = gsum_ref[...]
    cnt = jnp.clip(cnt_ref[0], 1.0)
    g1 = g[0] / cnt[:, None]
    g2 = g[1] / cnt[:, None]
    n1 = jnp.sqrt(jnp.sum(g1 * g1, axis=1, keepdims=True))
    n2 = jnp.sqrt(jnp.sum(g2 * g2, axis=1, keepdims=True))
    z1 = g1 / jnp.clip(n1, 1e-12)
    z2 = g2 / jnp.clip(n2, 1e-12)
    dn = (((1,), (1,)), ((), ()))
    s11 = lax.dot_general(z1, z1, dn, preferred_element_type=jnp.float32)
    s22 = lax.dot_general(z2, z2, dn, preferred_element_type=jnp.float32)
    s12 = lax.dot_general(z1, z2, dn, preferred_element_type=jnp.float32)
    e11 = jnp.exp(s11)
    e22 = jnp.exp(s22)
    e12 = jnp.exp(s12)
    r0 = lax.broadcasted_iota(jnp.int32, (GG, GG), 0)
    r1 = lax.broadcasted_iota(jnp.int32, (GG, GG), 1)
    eye = r0 == r1
    zeros = jnp.zeros((GG, GG), jnp.float32)
    d11 = jnp.sum(jnp.where(eye, e11, zeros), axis=1)
    d22 = jnp.sum(jnp.where(eye, e22, zeros), axis=1)
    d12 = jnp.sum(jnp.where(eye, s12, zeros), axis=1)
    x1 = jnp.sum(e11, axis=1) + jnp.sum(e12, axis=1) - d11
    x2 = jnp.sum(e22, axis=1) + jnp.sum(e12, axis=0) - d22
    loss = (jnp.log(x1) - d12) + (jnp.log(x2) - d12)
    o_ref[...] = jnp.mean(loss)[None, None]


_tanh_proj = pl.pallas_call(
    _tanh_proj_body,
    grid=(NBLK,),
    in_specs=[
        pl.BlockSpec((BLK, DD), lambda i: (i, 0)),
        pl.BlockSpec((DD, DD), lambda i: (0, 0)),
    ],
    out_specs=pl.BlockSpec((BLK, DD), lambda i: (i, 0)),
    out_shape=jax.ShapeDtypeStruct((NN, DD), jnp.float32),
)

_layer1 = pl.pallas_call(
    _layer_body,
    grid=(2, NBLK),
    in_specs=[
        pl.BlockSpec((1, BLK, DD), lambda c, i: (c, i, 0)),
        pl.BlockSpec((1, BLK, DD), lambda c, i: (c, i, 0)),
        pl.BlockSpec((BLK, DD), lambda c, i: (i, 0)),
        pl.BlockSpec((DD, DD), lambda c, i: (0, 0)),
    ],
    out_specs=pl.BlockSpec((BLK, DD), lambda c, i: (c * NBLK + i, 0)),
    out_shape=jax.ShapeDtypeStruct((2 * NN, DD), jnp.float32),
)

_layer2_readout = pl.pallas_call(
    _layer_readout_body,
    grid=(2, NBLK),
    in_specs=[
        pl.BlockSpec((1, BLK, DD), lambda c, i: (c, i, 0)),
        pl.BlockSpec((1, BLK, DD), lambda c, i: (c, i, 0)),
        pl.BlockSpec((BLK, DD), lambda c, i: (c * NBLK + i, 0)),
        pl.BlockSpec((DD, DD), lambda c, i: (0, 0)),
        pl.BlockSpec((1, 1, BLK), lambda c, i: (i, 0, 0)),
    ],
    out_specs=[
        pl.BlockSpec((1, GG, DD), lambda c, i: (c, 0, 0)),
        pl.BlockSpec((1, GG), lambda c, i: (0, 0)),
    ],
    out_shape=[
        jax.ShapeDtypeStruct((2, GG, DD), jnp.float32),
        jax.ShapeDtypeStruct((1, GG), jnp.float32),
    ],
)

_loss = pl.pallas_call(
    _loss_body,
    in_specs=[
        pl.BlockSpec((2, GG, DD), lambda: (0, 0, 0)),
        pl.BlockSpec((1, GG), lambda: (0, 0)),
    ],
    out_specs=pl.BlockSpec((1, 1), lambda: (0, 0)),
    out_shape=jax.ShapeDtypeStruct((1, 1), jnp.float32),
)


@jax.jit
def kernel(x, batch_vec, edge_index1, edge_index2, W_lm, W1, W2):
    src1 = edge_index1[0].astype(jnp.int32)
    dst1 = edge_index1[1].astype(jnp.int32)
    src2 = edge_index2[0].astype(jnp.int32)
    dst2 = edge_index2[1].astype(jnp.int32)
    npad_e = EPAD - EE
    pad0 = jnp.zeros((npad_e,), jnp.int32)
    padd = jnp.full((npad_e,), NPAD - 1, jnp.int32)
    srcs = jnp.stack([
        jnp.concatenate([src1, pad0]),
        jnp.concatenate([src2, pad0]),
    ])
    set_off = jnp.array([0, NN], jnp.int32)[:, None]
    dsts = jnp.stack([
        jnp.concatenate([dst1, padd]),
        jnp.concatenate([dst2, padd]),
    ])
    sd_a = jnp.stack([srcs.reshape(2, NSUB, CH, K),
                      dsts.reshape(2, NSUB, CH, K)], axis=3)
    sd_b = jnp.stack([(srcs + set_off).reshape(2, NSUB, CH, K),
                      dsts.reshape(2, NSUB, CH, K)], axis=3)
    dd = dsts.reshape(2, NSUB, CHD, KD)
    batch3d = batch_vec.astype(jnp.int32).reshape(NBLK, 1, BLK)

    deg = _make_sc_deg()(dd)
    h0 = _tanh_proj(x.astype(jnp.float32), W_lm)
    agg1 = _make_sc_agg("sc_agg_a")(h0, sd_a)
    h1 = _layer1(agg1, deg, h0, W1)
    agg2 = _make_sc_agg("sc_agg_b")(h1, sd_b)
    gsum, cnt = _layer2_readout(agg2, deg, h1, W2, batch3d)
    out = _loss(gsum, cnt)
    return out.reshape(())
